# baseline (speedup 1.0000x reference)
import functools

import jax
import jax.numpy as jnp
from jax import lax
from jax.experimental import pallas as pl
from jax.experimental.pallas import tpu as pltpu
from jax.experimental.pallas import tpu_sc as plsc

NC, NS, L = 2, 16, 16
NW = NC * NS
CH = 128


GRP = 4


def _tile_rows(E):
    nchk = E // CH
    base = nchk // NW
    extra = nchk - base * NW
    return nchk, base, extra


def _row_start(wid, base, extra):
    return jnp.where(
        wid < extra, (base + 1) * wid, extra * (base + 1) + base * (wid - extra)
    )


@functools.lru_cache(maxsize=None)
def _gather2(N, D, E, dtype=jnp.bfloat16):
    nchk, base, extra = _tile_rows(E)
    maxrows = base + (1 if extra else 0)
    nfullg = base // GRP
    rem = base - nfullg * GRP
    mesh = plsc.VectorSubcoreMesh(core_axis_name="c", subcore_axis_name="s")

    @functools.partial(
        pl.kernel,
        mesh=mesh,
        out_type=(
            jax.ShapeDtypeStruct((E, D), dtype),
            jax.ShapeDtypeStruct((E, D), dtype),
        ),
        scratch_types=[
            pltpu.VMEM((maxrows, CH), jnp.int32),
            pltpu.VMEM((GRP * CH, D), dtype),
            pltpu.VMEM((GRP * CH, D), dtype),
            pltpu.SemaphoreType.DMA,
            pltpu.SemaphoreType.DMA,
        ],
        compiler_params=pltpu.CompilerParams(use_tc_tiling_on_sc=False),
    )
    def k(table_hbm, row_hbm, col_hbm, xr_hbm, xc_hbm, idx_v, bufa, bufb, gsem, osem):
        wid = lax.axis_index("s") * NC + lax.axis_index("c")
        row0 = _row_start(wid, base, extra)
        has_extra = wid < extra
        bufs = (bufa, bufb)

        def do(idx2d_hbm, out_hbm):
            pltpu.sync_copy(
                idx2d_hbm.at[pl.ds(row0, base)], idx_v.at[pl.ds(0, base)]
            )
            if extra:
                @pl.when(has_extra)
                def _():
                    pltpu.sync_copy(
                        idx2d_hbm.at[pl.ds(row0 + base, 1)],
                        idx_v.at[pl.ds(base, 1)],
                    )

            groups = []
            for g in range(nfullg):
                groups.append((g * GRP, GRP, False))
            if rem or extra:
                groups.append((nfullg * GRP, rem, bool(extra)))

            out_desc = [None, None]
            for gi, (r0, nr, cond) in enumerate(groups):
                buf = bufs[gi % 2]
                if out_desc[gi % 2] is not None:
                    out_desc[gi % 2].wait()
                descs = []
                for j in range(nr):
                    descs.append(
                        pltpu.async_copy(
                            table_hbm.at[idx_v.at[r0 + j]],
                            buf.at[pl.ds(j * CH, CH)],
                            gsem,
                        )
                    )
                if cond:
                    @pl.when(has_extra)
                    def _(r0=r0, nr=nr, buf=buf):
                        pltpu.async_copy(
                            table_hbm.at[idx_v.at[r0 + nr]],
                            buf.at[pl.ds(nr * CH, CH)],
                            gsem,
                        ).wait()
                for d in descs:
                    d.wait()
                off = (row0 + r0) * CH
                out_desc[gi % 2] = pltpu.async_copy(
                    buf.at[pl.ds(0, nr * CH)], out_hbm.at[pl.ds(off, nr * CH)], osem
                )
                if cond:
                    @pl.when(has_extra)
                    def _(r0=r0, nr=nr, buf=buf):
                        pltpu.async_copy(
                            buf.at[pl.ds(nr * CH, CH)],
                            out_hbm.at[pl.ds((row0 + r0 + nr) * CH, CH)],
                            osem,
                        ).wait()
            for d in out_desc:
                if d is not None:
                    d.wait()

        do(row_hbm, xr_hbm)
        do(col_hbm, xc_hbm)

    return k


@functools.lru_cache(maxsize=None)
def _scatter_add(E, D, N):
    nchk, base, extra = _tile_rows(E)
    maxrows = base + (1 if extra else 0)
    nfullg = base // GRP
    rem = base - nfullg * GRP
    RPT = N // NS
    mesh = plsc.VectorSubcoreMesh(core_axis_name="c", subcore_axis_name="s")

    @functools.partial(
        pl.kernel,
        mesh=mesh,
        out_type=jax.ShapeDtypeStruct((NC, N, D), jnp.float32),
        scratch_types=[
            pltpu.VMEM((maxrows, CH), jnp.int32),
            pltpu.VMEM((GRP * CH, D), jnp.float32),
            pltpu.VMEM((GRP * CH, D), jnp.float32),
            pltpu.VMEM_SHARED((N, D), jnp.float32),
            pltpu.SemaphoreType.DMA,
        ],
        compiler_params=pltpu.CompilerParams(use_tc_tiling_on_sc=False),
    )
    def k(vals_hbm, idx_hbm, zeros_hbm, out_hbm, idx_v, bufa, bufb, acc, lsem):
        cid = lax.axis_index("c")
        sid = lax.axis_index("s")
        wid = sid * NC + cid
        row0 = _row_start(wid, base, extra)
        has_extra = wid < extra
        bufs = (bufa, bufb)

        pltpu.sync_copy(
            zeros_hbm.at[pl.ds(sid * RPT, RPT)], acc.at[pl.ds(sid * RPT, RPT)]
        )
        pltpu.sync_copy(idx_hbm.at[pl.ds(row0, base)], idx_v.at[pl.ds(0, base)])
        if extra:
            @pl.when(has_extra)
            def _():
                pltpu.sync_copy(
                    idx_hbm.at[pl.ds(row0 + base, 1)], idx_v.at[pl.ds(base, 1)]
                )
        plsc.subcore_barrier()

        groups = []
        for g in range(nfullg):
            groups.append((g * GRP, GRP, False))
        if rem or extra:
            groups.append((nfullg * GRP, rem, bool(extra)))

        def load(r0, nr, cond, buf):
            d = pltpu.async_copy(
                vals_hbm.at[pl.ds((row0 + r0) * CH, nr * CH)],
                buf.at[pl.ds(0, nr * CH)],
                lsem,
            )
            dx = [d]
            if cond:
                @pl.when(has_extra)
                def _():
                    pltpu.async_copy(
                        vals_hbm.at[pl.ds((row0 + r0 + nr) * CH, CH)],
                        buf.at[pl.ds(nr * CH, CH)],
                        lsem,
                    ).wait()
            return dx

        descs = {}
        descs[0] = load(*groups[0], bufs[0])
        if len(groups) > 1:
            descs[1] = load(*groups[1], bufs[1])
        for gi, (r0, nr, cond) in enumerate(groups):
            buf = bufs[gi % 2]
            for d in descs.pop(gi):
                d.wait()
            for j in range(nr):
                pltpu.sync_copy(
                    buf.at[pl.ds(j * CH, CH)],
                    acc.at[idx_v.at[r0 + j]],
                    add=True,
                )
            if cond:
                @pl.when(has_extra)
                def _(r0=r0, nr=nr, buf=buf):
                    pltpu.sync_copy(
                        buf.at[pl.ds(nr * CH, CH)],
                        acc.at[idx_v.at[r0 + nr]],
                        add=True,
                    )
            if gi + 2 < len(groups):
                descs[gi + 2] = load(*groups[gi + 2], buf)

        plsc.subcore_barrier()
        pltpu.sync_copy(
            acc.at[pl.ds(sid * RPT, RPT)], out_hbm.at[cid, pl.ds(sid * RPT, RPT)]
        )

    return k


_BE = 4000
_BN = 2000


def _full(shape):
    return pl.BlockSpec(shape, lambda i: (0,) * len(shape))


def _edge_body(hw, dx, xr_r, xc_r, ea_r, G1, bz1, G2, be2, bh, W2n, b2n,
               ea2_r, h_r):
    bf = jnp.bfloat16
    f32 = jnp.float32
    xcat = jnp.concatenate(
        [xr_r[...][:, 0:dx], xc_r[...][:, 0:dx], ea_r[...].astype(bf)],
        axis=1,
    )
    Z = jax.lax.dot(xcat, G1[...], preferred_element_type=f32) + bz1[...]
    t = jnp.maximum(Z[:, 0:64], 0.0).astype(bf)
    P = jax.lax.dot(t, G2[...], preferred_element_type=f32)
    ea2_r[...] = P[:, 0:64] + be2[...]
    t2 = jnp.maximum(P[:, 64:128] + Z[:, 64:128] + bh[...], 0.0).astype(bf)
    h = jax.lax.dot(t2, W2n[...], preferred_element_type=f32) + b2n[...]
    if hw > 64:
        h = jnp.concatenate([h, jnp.ones((h.shape[0], hw - 64), f32)], axis=1)
    h_r[...] = h


def _tc_edge(xr, xc, ea, G1, bz1, G2, be2, bh, W2n, b2n, dx, hw=64):
    E, Dg = xr.shape
    De = ea.shape[1]
    G = E // _BE
    grid_spec = pl.GridSpec(
        grid=(G,),
        in_specs=[
            pl.BlockSpec((_BE, Dg), lambda i: (i, 0)),
            pl.BlockSpec((_BE, Dg), lambda i: (i, 0)),
            pl.BlockSpec((_BE, De), lambda i: (i, 0)),
            _full(G1.shape), _full(bz1.shape), _full(G2.shape),
            _full(be2.shape), _full(bh.shape), _full(W2n.shape),
            _full(b2n.shape),
        ],
        out_specs=[
            pl.BlockSpec((_BE, 64), lambda i: (i, 0)),
            pl.BlockSpec((_BE, hw), lambda i: (i, 0)),
        ],
    )
    return pl.pallas_call(
        functools.partial(_edge_body, hw, dx),
        grid_spec=grid_spec,
        out_shape=[
            jax.ShapeDtypeStruct((E, 64), jnp.float32),
            jax.ShapeDtypeStruct((E, hw), jnp.float32),
        ],
        compiler_params=pltpu.CompilerParams(
            dimension_semantics=("arbitrary",)
        ),
    )(xr, xc, ea, G1, bz1, G2, be2, bh, W2n, b2n)


def _edge_final_body(xr_r, xc_r, ea_r, W1, b1, W2, b2, out_r):
    bf = jnp.bfloat16
    xcat = jnp.concatenate(
        [xr_r[...][:, 0:64], xc_r[...][:, 0:64], ea_r[...].astype(bf)],
        axis=1,
    )
    t = jax.lax.dot(xcat, W1[...], preferred_element_type=jnp.float32) + b1[...]
    t = jnp.maximum(t, 0.0).astype(bf)
    z = jax.lax.dot(t, W2[...], preferred_element_type=jnp.float32) + b2[...]
    out_r[...] = jax.nn.sigmoid(z)


def _tc_edge_final(xr, xc, ea, W1, b1, W2, b2):
    E, Dg = xr.shape
    De = ea.shape[1]
    G = E // _BE
    grid_spec = pl.GridSpec(
        grid=(G,),
        in_specs=[
            pl.BlockSpec((_BE, Dg), lambda i: (i, 0)),
            pl.BlockSpec((_BE, Dg), lambda i: (i, 0)),
            pl.BlockSpec((_BE, De), lambda i: (i, 0)),
            _full(W1.shape), _full(b1.shape),
            _full(W2.shape), _full(b2.shape),
        ],
        out_specs=[pl.BlockSpec((_BE, 1), lambda i: (i, 0))],
    )
    return pl.pallas_call(
        _edge_final_body,
        grid_spec=grid_spec,
        out_shape=[jax.ShapeDtypeStruct((E, 1), jnp.float32)],
        compiler_params=pltpu.CompilerParams(
            dimension_semantics=("arbitrary",)
        ),
    )(xr, xc, ea, W1, b1, W2, b2)[0]


def _node_body(ds, x_r, sp_r, cp_r, V1, b1, W2, b2, out_r, outg_r):
    bf = jnp.bfloat16
    if ds > 64:
        s = sp_r[0, :, 0:64] + sp_r[1, :, 0:64]
        c = sp_r[0, :, 64:65] + sp_r[1, :, 64:65]
    else:
        s = sp_r[0] + sp_r[1]
        c = cp_r[0, :, 0:1] + cp_r[1, :, 0:1]
    agg = s / jnp.maximum(c, 1.0)
    xcat = jnp.concatenate([x_r[...].astype(bf), agg.astype(bf)], axis=1)
    t = jax.lax.dot(xcat, V1[...], preferred_element_type=jnp.float32) + b1[...]
    t = jnp.maximum(t, 0.0).astype(bf)
    o = jax.lax.dot(t, W2[...], preferred_element_type=jnp.float32) + b2[...]
    out_r[...] = o
    outg_r[...] = jnp.concatenate(
        [o.astype(bf), jnp.zeros((o.shape[0], 64), bf)], axis=1
    )


def _tc_node(x, spart, cpart, V1, b1, W2, b2):
    N, Dx = x.shape
    ds = spart.shape[2]
    G = N // _BN
    grid_spec = pl.GridSpec(
        grid=(G,),
        in_specs=[
            pl.BlockSpec((_BN, Dx), lambda i: (i, 0)),
            pl.BlockSpec((NC, _BN, ds), lambda i: (0, i, 0)),
            pl.BlockSpec((NC, _BN, 16), lambda i: (0, i, 0)),
            _full(V1.shape), _full(b1.shape),
            _full(W2.shape), _full(b2.shape),
        ],
        out_specs=[
            pl.BlockSpec((_BN, 64), lambda i: (i, 0)),
            pl.BlockSpec((_BN, 128), lambda i: (i, 0)),
        ],
    )
    return pl.pallas_call(
        functools.partial(_node_body, ds),
        grid_spec=grid_spec,
        out_shape=[
            jax.ShapeDtypeStruct((N, 64), jnp.float32),
            jax.ShapeDtypeStruct((N, 128), jnp.bfloat16),
        ],
        compiler_params=pltpu.CompilerParams(
            dimension_semantics=("arbitrary",)
        ),
    )(x, spart, cpart, V1, b1, W2, b2)


def _row(b):
    return b.reshape(1, -1)


def _pad_rows(W, rows_from, n_rows):
    sub = W[rows_from[0]:rows_from[1]]
    return jnp.pad(sub, ((0, n_rows - sub.shape[0]), (0, 0)))


def _prep_edge(pe, pn1, dx_raw, dx):
    bf = jnp.bfloat16
    W1e, b1e, W2e, b2e = pe
    W1n, b1n, W2n, b2n = pn1
    de_raw = W1e.shape[0] - 2 * dx_raw
    if dx_raw == dx:
        A, B, C = W1e[0:dx], W1e[dx:2 * dx], W1e[2 * dx:]
    else:
        A = _pad_rows(W1e, (0, dx_raw), dx)
        B = _pad_rows(W1e, (dx_raw, 2 * dx_raw), dx)
        C = _pad_rows(W1e, (2 * dx_raw, 2 * dx_raw + de_raw), dx)
    D1 = W1n[0:dx_raw]
    if dx_raw != dx:
        D1 = _pad_rows(W1n, (0, dx_raw), dx)
    Ew = W1n[dx_raw:dx_raw + 64]
    G1 = jnp.concatenate(
        [jnp.concatenate([A, B, C], axis=0),
         jnp.concatenate([D1, jnp.zeros((2 * dx, 64), jnp.float32)], axis=0)],
        axis=1).astype(bf)
    bz1 = jnp.concatenate([b1e, jnp.zeros((64,), jnp.float32)]).reshape(1, 128)
    W2eEw = W2e @ Ew
    G2 = jnp.concatenate([W2e, W2eEw], axis=1).astype(bf)
    bh = _row(b2e @ Ew + b1n)
    return G1, bz1, G2, _row(b2e), bh, W2n.astype(bf), _row(b2n)


def _prep_node(pn2, dx_raw, dx):
    bf = jnp.bfloat16
    V1n, c1n, V2n, c2n = pn2
    Wx = V1n[0:dx_raw]
    if dx_raw != dx:
        Wx = _pad_rows(V1n, (0, dx_raw), dx)
    Wa = V1n[dx_raw:dx_raw + 64]
    V1 = jnp.concatenate([Wx, Wa], axis=0).astype(bf)
    return V1, _row(c1n), V2n.astype(bf), _row(c2n)


def kernel(x, edge_index, edge_attr, params):
    N, _ = x.shape
    E = edge_attr.shape[0]
    row = edge_index[0].reshape(E // CH, CH)
    col = edge_index[1].reshape(E // CH, CH)

    bf = jnp.bfloat16
    x16 = jnp.pad(x, ((0, 0), (0, 14)))
    xg = jnp.pad(x, ((0, 0), (0, 126))).astype(bf)
    ea16 = jnp.pad(edge_attr, ((0, 0), (0, 12)))
    zeros64 = jnp.zeros((N, 64), jnp.float32)
    zeros16 = jnp.zeros((N, 16), jnp.float32)
    ones16 = jnp.ones((E, 16), jnp.float32)

    cpart = _scatter_add(E, 16, N)(ones16, col, zeros16)

    ew = _prep_edge(params['c1_e'], params['c1_n1'], 2, 16)
    xr, xc = _gather2(N, 128, E)(xg, row, col)
    ea, h = _tc_edge(xr, xc, ea16, *ew, dx=16)
    spart = _scatter_add(E, 64, N)(h, col, zeros64)
    nw = _prep_node(params['c1_n2'], 2, 16)
    xl, xlg = _tc_node(x16, spart, cpart, *nw)

    for name in ('c2', 'c3', 'c4'):
        ew = _prep_edge(params[name + '_e'], params[name + '_n1'], 64, 64)
        xr, xc = _gather2(N, 128, E)(xlg, row, col)
        ea, h = _tc_edge(xr, xc, ea, *ew, dx=64)
        spart = _scatter_add(E, 64, N)(h, col, zeros64)
        nw = _prep_node(params[name + '_n2'], 64, 64)
        xl, xlg = _tc_node(xl, spart, cpart, *nw)

    W1f, b1f, W2f, b2f = params['c5_e']
    xr, xc = _gather2(N, 128, E)(xlg, row, col)
    return _tc_edge_final(xr, xc, ea, W1f.astype(bf), _row(b1f),
                          W2f.astype(bf), _row(b2f).reshape(1, 1))


# speedup vs baseline: 1.7383x; 1.7383x over previous
import functools

import jax
import jax.numpy as jnp
from jax import lax
from jax.experimental import pallas as pl
from jax.experimental.pallas import tpu as pltpu
from jax.experimental.pallas import tpu_sc as plsc

NC, NS, L = 2, 16, 16
NW = NC * NS
CH = 128


GRP = 3


def _tile_rows(E):
    nchk = E // CH
    base = nchk // NW
    extra = nchk - base * NW
    return nchk, base, extra


def _idx_pad_rows(E):
    nchk, base, extra = _tile_rows(E)
    maxrow0 = base * (NW - 1) + extra
    return (maxrow0 // 8) * 8 + (base + (1 if extra else 0)) + 8


def _row_start(wid, base, extra):
    return jnp.where(
        wid < extra, (base + 1) * wid, extra * (base + 1) + base * (wid - extra)
    )


@functools.lru_cache(maxsize=None)
def _gather2(N, D, E, dtype=jnp.float32):
    nchk, base, extra = _tile_rows(E)
    maxrows = base + (1 if extra else 0)
    nfullg = base // GRP
    rem = base - nfullg * GRP
    pre = maxrows + 8
    nchkp = _idx_pad_rows(E)
    mesh = plsc.VectorSubcoreMesh(core_axis_name="c", subcore_axis_name="s")

    @functools.partial(
        pl.kernel,
        mesh=mesh,
        out_type=(
            jax.ShapeDtypeStruct((E, D), dtype),
            jax.ShapeDtypeStruct((E, D), dtype),
        ),
        scratch_types=[
            pltpu.VMEM((pre, CH), jnp.int32),
            pltpu.VMEM((GRP * CH, D), dtype),
            pltpu.VMEM((GRP * CH, D), dtype),
            pltpu.SemaphoreType.DMA,
            pltpu.SemaphoreType.DMA,
        ],
    )
    def k(table_hbm, row_hbm, col_hbm, xr_hbm, xc_hbm, idx_v, bufa, bufb, gsem, osem):
        wid = lax.axis_index("s") * NC + lax.axis_index("c")
        row0 = _row_start(wid, base, extra)
        row0a = pl.multiple_of((row0 // 8) * 8, 8)
        delta = row0 - row0a
        has_extra = wid < extra
        bufs = (bufa, bufb)

        def do(idx2d_hbm, out_hbm):
            pltpu.sync_copy(idx2d_hbm.at[pl.ds(row0a, pre)], idx_v)

            groups = []
            for g in range(nfullg):
                groups.append((g * GRP, GRP, False))
            if rem or extra:
                groups.append((nfullg * GRP, rem, bool(extra)))

            out_desc = [None, None]
            for gi, (r0, nr, cond) in enumerate(groups):
                buf = bufs[gi % 2]
                if out_desc[gi % 2] is not None and nr:
                    out_desc[gi % 2].wait()
                descs = []
                for j in range(nr):
                    descs.append(
                        pltpu.async_copy(
                            table_hbm.at[idx_v.at[delta + (r0 + j)]],
                            buf.at[pl.ds(j * CH, CH)],
                            gsem,
                        )
                    )
                if cond:
                    @pl.when(has_extra)
                    def _(r0=r0, nr=nr, buf=buf):
                        pltpu.async_copy(
                            table_hbm.at[idx_v.at[delta + (r0 + nr)]],
                            buf.at[pl.ds(nr * CH, CH)],
                            gsem,
                        ).wait()
                for d in descs:
                    d.wait()
                off = (row0 + r0) * CH
                if nr:
                    out_desc[gi % 2] = pltpu.async_copy(
                        buf.at[pl.ds(0, nr * CH)],
                        out_hbm.at[pl.ds(off, nr * CH)],
                        osem,
                    )
                if cond:
                    @pl.when(has_extra)
                    def _(r0=r0, nr=nr, buf=buf):
                        pltpu.async_copy(
                            buf.at[pl.ds(nr * CH, CH)],
                            out_hbm.at[pl.ds((row0 + r0 + nr) * CH, CH)],
                            osem,
                        ).wait()
            for d in out_desc:
                if d is not None:
                    d.wait()

        do(row_hbm, xr_hbm)
        do(col_hbm, xc_hbm)

    return k


@functools.lru_cache(maxsize=None)
def _scatter_add(E, D, N):
    nchk, base, extra = _tile_rows(E)
    maxrows = base + (1 if extra else 0)
    nfullg = base // GRP
    rem = base - nfullg * GRP
    pre = maxrows + 8
    ZR = (N // NS // 8) * 8
    ZLAST = N - ZR * (NS - 1)
    mesh = plsc.VectorSubcoreMesh(core_axis_name="c", subcore_axis_name="s")

    @functools.partial(
        pl.kernel,
        mesh=mesh,
        out_type=jax.ShapeDtypeStruct((NC, N, D), jnp.float32),
        scratch_types=[
            pltpu.VMEM((pre, CH), jnp.int32),
            pltpu.VMEM((GRP * CH, D), jnp.float32),
            pltpu.VMEM((GRP * CH, D), jnp.float32),
            pltpu.VMEM_SHARED((N, D), jnp.float32),
            pltpu.SemaphoreType.DMA,
        ],
        compiler_params=pltpu.CompilerParams(use_tc_tiling_on_sc=False),
    )
    def k(vals_hbm, idx_hbm, zeros_hbm, out_hbm, idx_v, bufa, bufb, acc, lsem):
        cid = lax.axis_index("c")
        sid = lax.axis_index("s")
        wid = sid * NC + cid
        row0 = _row_start(wid, base, extra)
        row0a = pl.multiple_of((row0 // 8) * 8, 8)
        delta = row0 - row0a
        has_extra = wid < extra
        bufs = (bufa, bufb)

        z0 = pl.multiple_of(sid * ZR, 8)
        zn = jnp.where(sid == NS - 1, ZLAST, ZR)

        @pl.when(sid < NS - 1)
        def _():
            pltpu.sync_copy(
                zeros_hbm.at[pl.ds(0, ZR)], acc.at[pl.ds(z0, ZR)]
            )

        @pl.when(sid == NS - 1)
        def _():
            pltpu.sync_copy(zeros_hbm, acc.at[pl.ds(z0, ZLAST)])

        pltpu.sync_copy(idx_hbm.at[pl.ds(row0a, pre)], idx_v)
        plsc.subcore_barrier()

        groups = []
        for g in range(nfullg):
            groups.append((g * GRP, GRP, False))
        if rem or extra:
            groups.append((nfullg * GRP, rem, bool(extra)))

        def load(r0, nr, cond, buf):
            dx = []
            if nr:
                dx.append(pltpu.async_copy(
                    vals_hbm.at[pl.ds((row0 + r0) * CH, nr * CH)],
                    buf.at[pl.ds(0, nr * CH)],
                    lsem,
                ))
            if cond:
                @pl.when(has_extra)
                def _():
                    pltpu.async_copy(
                        vals_hbm.at[pl.ds((row0 + r0 + nr) * CH, CH)],
                        buf.at[pl.ds(nr * CH, CH)],
                        lsem,
                    ).wait()
            return dx

        descs = {}
        descs[0] = load(*groups[0], bufs[0])
        if len(groups) > 1:
            descs[1] = load(*groups[1], bufs[1])
        for gi, (r0, nr, cond) in enumerate(groups):
            buf = bufs[gi % 2]
            for d in descs.pop(gi):
                d.wait()
            for j in range(nr):
                pltpu.sync_copy(
                    buf.at[pl.ds(j * CH, CH)],
                    acc.at[idx_v.at[delta + (r0 + j)]],
                    add=True,
                )
            if cond:
                @pl.when(has_extra)
                def _(r0=r0, nr=nr, buf=buf):
                    pltpu.sync_copy(
                        buf.at[pl.ds(nr * CH, CH)],
                        acc.at[idx_v.at[delta + (r0 + nr)]],
                        add=True,
                    )
            if gi + 2 < len(groups):
                descs[gi + 2] = load(*groups[gi + 2], buf)

        plsc.subcore_barrier()

        @pl.when(sid < NS - 1)
        def _():
            pltpu.sync_copy(
                acc.at[pl.ds(z0, ZR)], out_hbm.at[cid, pl.ds(z0, ZR)]
            )

        @pl.when(sid == NS - 1)
        def _():
            pltpu.sync_copy(
                acc.at[pl.ds(z0, ZLAST)], out_hbm.at[cid, pl.ds(z0, ZLAST)]
            )

    return k


_BE = 4000
_BN = 2000


def _full(shape):
    return pl.BlockSpec(shape, lambda i: (0,) * len(shape))


def _edge_body(dx, xr_r, xc_r, ea_r, G1, bz1, G2, be2, bh, W2n, b2n,
               ea2_r, h_r):
    bf = jnp.bfloat16
    f32 = jnp.float32
    xcat = jnp.concatenate(
        [xr_r[...][:, 0:dx].astype(bf), xc_r[...][:, 0:dx].astype(bf),
         ea_r[...].astype(bf)],
        axis=1,
    )
    Z = jax.lax.dot(xcat, G1[...], preferred_element_type=f32) + bz1[...]
    t = jnp.maximum(Z[:, 0:64], 0.0).astype(bf)
    P = jax.lax.dot(t, G2[...], preferred_element_type=f32)
    ea2_r[...] = P[:, 0:64] + be2[...]
    t2 = jnp.maximum(P[:, 64:128] + Z[:, 64:128] + bh[...], 0.0).astype(bf)
    h_r[...] = jax.lax.dot(t2, W2n[...], preferred_element_type=f32) + b2n[...]


def _tc_edge(xr, xc, ea, G1, bz1, G2, be2, bh, W2n, b2n, dx):
    E, Dg = xr.shape
    De = ea.shape[1]
    G = E // _BE
    grid_spec = pl.GridSpec(
        grid=(G,),
        in_specs=[
            pl.BlockSpec((_BE, Dg), lambda i: (i, 0)),
            pl.BlockSpec((_BE, Dg), lambda i: (i, 0)),
            pl.BlockSpec((_BE, De), lambda i: (i, 0)),
            _full(G1.shape), _full(bz1.shape), _full(G2.shape),
            _full(be2.shape), _full(bh.shape), _full(W2n.shape),
            _full(b2n.shape),
        ],
        out_specs=[
            pl.BlockSpec((_BE, 64), lambda i: (i, 0)),
            pl.BlockSpec((_BE, 64), lambda i: (i, 0)),
        ],
    )
    return pl.pallas_call(
        functools.partial(_edge_body, dx),
        grid_spec=grid_spec,
        out_shape=[
            jax.ShapeDtypeStruct((E, 64), jnp.float32),
            jax.ShapeDtypeStruct((E, 64), jnp.float32),
        ],
        compiler_params=pltpu.CompilerParams(
            dimension_semantics=("arbitrary",)
        ),
    )(xr, xc, ea, G1, bz1, G2, be2, bh, W2n, b2n)


def _edge_final_body(xr_r, xc_r, ea_r, W1, b1, W2, b2, out_r):
    bf = jnp.bfloat16
    xcat = jnp.concatenate(
        [xr_r[...][:, 0:64], xc_r[...][:, 0:64], ea_r[...].astype(bf)],
        axis=1,
    )
    t = jax.lax.dot(xcat, W1[...], preferred_element_type=jnp.float32) + b1[...]
    t = jnp.maximum(t, 0.0).astype(bf)
    z = jax.lax.dot(t, W2[...], preferred_element_type=jnp.float32) + b2[...]
    out_r[...] = jax.nn.sigmoid(z)


def _tc_edge_final(xr, xc, ea, W1, b1, W2, b2):
    E, Dg = xr.shape
    De = ea.shape[1]
    G = E // _BE
    grid_spec = pl.GridSpec(
        grid=(G,),
        in_specs=[
            pl.BlockSpec((_BE, Dg), lambda i: (i, 0)),
            pl.BlockSpec((_BE, Dg), lambda i: (i, 0)),
            pl.BlockSpec((_BE, De), lambda i: (i, 0)),
            _full(W1.shape), _full(b1.shape),
            _full(W2.shape), _full(b2.shape),
        ],
        out_specs=[pl.BlockSpec((_BE, 1), lambda i: (i, 0))],
    )
    return pl.pallas_call(
        _edge_final_body,
        grid_spec=grid_spec,
        out_shape=[jax.ShapeDtypeStruct((E, 1), jnp.float32)],
        compiler_params=pltpu.CompilerParams(
            dimension_semantics=("arbitrary",)
        ),
    )(xr, xc, ea, W1, b1, W2, b2)[0]


def _node_body(x_r, sp_r, cp_r, V1, b1, W2, b2, out_r, outg_r):
    bf = jnp.bfloat16
    f32 = jnp.float32
    s = sp_r[0] + sp_r[1]
    c = cp_r[0, :, 0:1] + cp_r[1, :, 0:1]
    agg = s / jnp.maximum(c, 1.0)
    xcat = jnp.concatenate([x_r[...].astype(bf), agg.astype(bf)], axis=1)
    t = jax.lax.dot(xcat, V1[...], preferred_element_type=f32) + b1[...]
    t = jnp.maximum(t, 0.0).astype(bf)
    o = jax.lax.dot(t, W2[...], preferred_element_type=f32) + b2[...]
    out_r[...] = o
    outg_r[...] = jnp.concatenate(
        [o, jnp.zeros((o.shape[0], 64), f32)], axis=1
    )


def _tc_node(x, spart, cpart, V1, b1, W2, b2):
    N, Dx = x.shape
    G = N // _BN
    grid_spec = pl.GridSpec(
        grid=(G,),
        in_specs=[
            pl.BlockSpec((_BN, Dx), lambda i: (i, 0)),
            pl.BlockSpec((NC, _BN, 64), lambda i: (0, i, 0)),
            pl.BlockSpec((NC, _BN, 16), lambda i: (0, i, 0)),
            _full(V1.shape), _full(b1.shape),
            _full(W2.shape), _full(b2.shape),
        ],
        out_specs=[
            pl.BlockSpec((_BN, 64), lambda i: (i, 0)),
            pl.BlockSpec((_BN, 128), lambda i: (i, 0)),
        ],
    )
    return pl.pallas_call(
        _node_body,
        grid_spec=grid_spec,
        out_shape=[
            jax.ShapeDtypeStruct((N, 64), jnp.float32),
            jax.ShapeDtypeStruct((N, 128), jnp.float32),
        ],
        compiler_params=pltpu.CompilerParams(
            dimension_semantics=("arbitrary",)
        ),
    )(x, spart, cpart, V1, b1, W2, b2)


def _row(b):
    return b.reshape(1, -1)


def _pad_rows(W, rows_from, n_rows):
    sub = W[rows_from[0]:rows_from[1]]
    return jnp.pad(sub, ((0, n_rows - sub.shape[0]), (0, 0)))


def _prep_edge(pe, pn1, dx_raw, dx):
    bf = jnp.bfloat16
    W1e, b1e, W2e, b2e = pe
    W1n, b1n, W2n, b2n = pn1
    de_raw = W1e.shape[0] - 2 * dx_raw
    if dx_raw == dx:
        A, B, C = W1e[0:dx], W1e[dx:2 * dx], W1e[2 * dx:]
    else:
        A = _pad_rows(W1e, (0, dx_raw), dx)
        B = _pad_rows(W1e, (dx_raw, 2 * dx_raw), dx)
        C = _pad_rows(W1e, (2 * dx_raw, 2 * dx_raw + de_raw), dx)
    D1 = W1n[0:dx_raw]
    if dx_raw != dx:
        D1 = _pad_rows(W1n, (0, dx_raw), dx)
    Ew = W1n[dx_raw:dx_raw + 64]
    G1 = jnp.concatenate(
        [jnp.concatenate([A, B, C], axis=0),
         jnp.concatenate([D1, jnp.zeros((2 * dx, 64), jnp.float32)], axis=0)],
        axis=1).astype(bf)
    bz1 = jnp.concatenate([b1e, jnp.zeros((64,), jnp.float32)]).reshape(1, 128)
    W2eEw = W2e @ Ew
    G2 = jnp.concatenate([W2e, W2eEw], axis=1).astype(bf)
    bh = _row(b2e @ Ew + b1n)
    return G1, bz1, G2, _row(b2e), bh, W2n.astype(bf), _row(b2n)


def _prep_node(pn2, dx_raw, dx):
    bf = jnp.bfloat16
    V1n, c1n, V2n, c2n = pn2
    Wx = V1n[0:dx_raw]
    if dx_raw != dx:
        Wx = _pad_rows(V1n, (0, dx_raw), dx)
    Wa = V1n[dx_raw:dx_raw + 64]
    V1 = jnp.concatenate([Wx, Wa], axis=0).astype(bf)
    return V1, _row(c1n), V2n.astype(bf), _row(c2n)


def kernel(x, edge_index, edge_attr, params):
    N, _ = x.shape
    E = edge_attr.shape[0]
    npad = _idx_pad_rows(E) - E // CH
    row = jnp.pad(edge_index[0].reshape(E // CH, CH), ((0, npad), (0, 0)))
    col = jnp.pad(edge_index[1].reshape(E // CH, CH), ((0, npad), (0, 0)))


    bf = jnp.bfloat16
    x16 = jnp.pad(x, ((0, 0), (0, 14)))
    xg = jnp.pad(x, ((0, 0), (0, 126)))
    ea16 = jnp.pad(edge_attr, ((0, 0), (0, 12)))
    zra = (N // NS // 8) * 8
    ztail = zra + N - zra * NS
    zeros64 = jnp.zeros((ztail, 64), jnp.float32)
    zeros16 = jnp.zeros((ztail, 16), jnp.float32)
    ones16 = jnp.ones((E, 16), jnp.float32)

    cpart = _scatter_add(E, 16, N)(ones16, col, zeros16)

    ew = _prep_edge(params['c1_e'], params['c1_n1'], 2, 16)
    xr, xc = _gather2(N, 128, E)(xg, row, col)
    ea, h = _tc_edge(xr, xc, ea16, *ew, dx=16)
    spart = _scatter_add(E, 64, N)(h, col, zeros64)
    nw = _prep_node(params['c1_n2'], 2, 16)
    xl, xlg = _tc_node(x16, spart, cpart, *nw)

    for name in ('c2', 'c3', 'c4'):
        ew = _prep_edge(params[name + '_e'], params[name + '_n1'], 64, 64)
        xr, xc = _gather2(N, 128, E)(xlg, row, col)
        ea, h = _tc_edge(xr, xc, ea, *ew, dx=64)
        spart = _scatter_add(E, 64, N)(h, col, zeros64)
        nw = _prep_node(params[name + '_n2'], 64, 64)
        xl, xlg = _tc_node(xl, spart, cpart, *nw)

    W1f, b1f, W2f, b2f = params['c5_e']
    xr, xc = _gather2(N, 128, E)(xlg, row, col)
    return _tc_edge_final(xr, xc, ea, W1f.astype(bf), _row(b1f),
                          W2f.astype(bf), _row(b2f).reshape(1, 1))


# speedup vs baseline: 1.8631x; 1.0718x over previous
import functools

import jax
import jax.numpy as jnp
from jax import lax
from jax.experimental import pallas as pl
from jax.experimental.pallas import tpu as pltpu
from jax.experimental.pallas import tpu_sc as plsc

NC, NS, L = 2, 16, 16
NW = NC * NS
CH = 128


GRP = 3


def _tile_rows(E):
    nchk = E // CH
    base = nchk // NW
    extra = nchk - base * NW
    return nchk, base, extra


def _idx_pad_rows(E):
    nchk, base, extra = _tile_rows(E)
    maxrow0 = base * (NW - 1) + extra
    return (maxrow0 // 8) * 8 + (base + (1 if extra else 0)) + 8


def _row_start(wid, base, extra):
    return jnp.where(
        wid < extra, (base + 1) * wid, extra * (base + 1) + base * (wid - extra)
    )


@functools.lru_cache(maxsize=None)
def _gather2(N, D, E, dtype=jnp.float32):
    nchk, base, extra = _tile_rows(E)
    maxrows = base + (1 if extra else 0)
    nfullg = base // GRP
    rem = base - nfullg * GRP
    pre = maxrows + 8
    nchkp = _idx_pad_rows(E)
    mesh = plsc.VectorSubcoreMesh(core_axis_name="c", subcore_axis_name="s")

    @functools.partial(
        pl.kernel,
        mesh=mesh,
        out_type=(
            jax.ShapeDtypeStruct((E, D), dtype),
            jax.ShapeDtypeStruct((E, D), dtype),
        ),
        scratch_types=[
            pltpu.VMEM((pre, CH), jnp.int32),
            pltpu.VMEM((GRP * CH, D), dtype),
            pltpu.VMEM((GRP * CH, D), dtype),
            pltpu.SemaphoreType.DMA,
            pltpu.SemaphoreType.DMA,
        ],
    )
    def k(table_hbm, row_hbm, col_hbm, xr_hbm, xc_hbm, idx_v, bufa, bufb, gsem, osem):
        wid = lax.axis_index("s") * NC + lax.axis_index("c")
        row0 = _row_start(wid, base, extra)
        row0a = pl.multiple_of((row0 // 8) * 8, 8)
        delta = row0 - row0a
        has_extra = wid < extra
        bufs = (bufa, bufb)

        def do(idx2d_hbm, out_hbm):
            pltpu.sync_copy(idx2d_hbm.at[pl.ds(row0a, pre)], idx_v)

            groups = []
            for g in range(nfullg):
                groups.append((g * GRP, GRP, False))
            if rem or extra:
                groups.append((nfullg * GRP, rem, bool(extra)))

            out_desc = [None, None]
            for gi, (r0, nr, cond) in enumerate(groups):
                buf = bufs[gi % 2]
                if out_desc[gi % 2] is not None and nr:
                    out_desc[gi % 2].wait()
                descs = []
                for j in range(nr):
                    descs.append(
                        pltpu.async_copy(
                            table_hbm.at[idx_v.at[delta + (r0 + j)]],
                            buf.at[pl.ds(j * CH, CH)],
                            gsem,
                        )
                    )
                if cond:
                    @pl.when(has_extra)
                    def _(r0=r0, nr=nr, buf=buf):
                        pltpu.async_copy(
                            table_hbm.at[idx_v.at[delta + (r0 + nr)]],
                            buf.at[pl.ds(nr * CH, CH)],
                            gsem,
                        ).wait()
                for d in descs:
                    d.wait()
                off = (row0 + r0) * CH
                if nr:
                    out_desc[gi % 2] = pltpu.async_copy(
                        buf.at[pl.ds(0, nr * CH)],
                        out_hbm.at[pl.ds(off, nr * CH)],
                        osem,
                    )
                if cond:
                    @pl.when(has_extra)
                    def _(r0=r0, nr=nr, buf=buf):
                        pltpu.async_copy(
                            buf.at[pl.ds(nr * CH, CH)],
                            out_hbm.at[pl.ds((row0 + r0 + nr) * CH, CH)],
                            osem,
                        ).wait()
            for d in out_desc:
                if d is not None:
                    d.wait()

        do(row_hbm, xr_hbm)
        do(col_hbm, xc_hbm)

    return k


@functools.lru_cache(maxsize=None)
def _scatter_add(E, D, N):
    nchk, base, extra = _tile_rows(E)
    maxrows = base + (1 if extra else 0)
    nfullg = base // GRP
    rem = base - nfullg * GRP
    pre = maxrows + 8
    ZR = (N // NS // 8) * 8
    ZLAST = N - ZR * (NS - 1)
    mesh = plsc.VectorSubcoreMesh(core_axis_name="c", subcore_axis_name="s")

    @functools.partial(
        pl.kernel,
        mesh=mesh,
        out_type=jax.ShapeDtypeStruct((NC, N, D), jnp.float32),
        scratch_types=[
            pltpu.VMEM((pre, CH), jnp.int32),
            pltpu.VMEM((GRP * CH, D), jnp.float32),
            pltpu.VMEM((GRP * CH, D), jnp.float32),
            pltpu.VMEM_SHARED((N, D), jnp.float32),
            pltpu.SemaphoreType.DMA,
        ],
        compiler_params=pltpu.CompilerParams(use_tc_tiling_on_sc=False),
    )
    def k(vals_hbm, idx_hbm, zeros_hbm, out_hbm, idx_v, bufa, bufb, acc, lsem):
        cid = lax.axis_index("c")
        sid = lax.axis_index("s")
        wid = sid * NC + cid
        row0 = _row_start(wid, base, extra)
        row0a = pl.multiple_of((row0 // 8) * 8, 8)
        delta = row0 - row0a
        has_extra = wid < extra
        bufs = (bufa, bufb)

        z0 = pl.multiple_of(sid * ZR, 8)
        zn = jnp.where(sid == NS - 1, ZLAST, ZR)

        @pl.when(sid < NS - 1)
        def _():
            pltpu.sync_copy(
                zeros_hbm.at[pl.ds(0, ZR)], acc.at[pl.ds(z0, ZR)]
            )

        @pl.when(sid == NS - 1)
        def _():
            pltpu.sync_copy(zeros_hbm, acc.at[pl.ds(z0, ZLAST)])

        pltpu.sync_copy(idx_hbm.at[pl.ds(row0a, pre)], idx_v)
        plsc.subcore_barrier()

        groups = []
        for g in range(nfullg):
            groups.append((g * GRP, GRP, False))
        if rem or extra:
            groups.append((nfullg * GRP, rem, bool(extra)))

        def load(r0, nr, cond, buf):
            dx = []
            if nr:
                dx.append(pltpu.async_copy(
                    vals_hbm.at[pl.ds((row0 + r0) * CH, nr * CH)],
                    buf.at[pl.ds(0, nr * CH)],
                    lsem,
                ))
            if cond:
                @pl.when(has_extra)
                def _():
                    pltpu.async_copy(
                        vals_hbm.at[pl.ds((row0 + r0 + nr) * CH, CH)],
                        buf.at[pl.ds(nr * CH, CH)],
                        lsem,
                    ).wait()
            return dx

        descs = {}
        descs[0] = load(*groups[0], bufs[0])
        if len(groups) > 1:
            descs[1] = load(*groups[1], bufs[1])
        for gi, (r0, nr, cond) in enumerate(groups):
            buf = bufs[gi % 2]
            for d in descs.pop(gi):
                d.wait()
            for j in range(nr):
                pltpu.sync_copy(
                    buf.at[pl.ds(j * CH, CH)],
                    acc.at[idx_v.at[delta + (r0 + j)]],
                    add=True,
                )
            if cond:
                @pl.when(has_extra)
                def _(r0=r0, nr=nr, buf=buf):
                    pltpu.sync_copy(
                        buf.at[pl.ds(nr * CH, CH)],
                        acc.at[idx_v.at[delta + (r0 + nr)]],
                        add=True,
                    )
            if gi + 2 < len(groups):
                descs[gi + 2] = load(*groups[gi + 2], buf)

        plsc.subcore_barrier()

        @pl.when(sid < NS - 1)
        def _():
            pltpu.sync_copy(
                acc.at[pl.ds(z0, ZR)], out_hbm.at[cid, pl.ds(z0, ZR)]
            )

        @pl.when(sid == NS - 1)
        def _():
            pltpu.sync_copy(
                acc.at[pl.ds(z0, ZLAST)], out_hbm.at[cid, pl.ds(z0, ZLAST)]
            )

    return k


_BE = 4000
_BN = 2000


def _full(shape):
    return pl.BlockSpec(shape, lambda i: (0,) * len(shape))


def _edge_body(dx, xr_r, xc_r, ea_r, G1, bz1, G2, be2, bh, W2n, b2n,
               ea2_r, h_r):
    bf = jnp.bfloat16
    f32 = jnp.float32
    xcat = jnp.concatenate(
        [xr_r[...][:, 0:dx].astype(bf), xc_r[...][:, 0:dx].astype(bf),
         ea_r[...].astype(bf)],
        axis=1,
    )
    Z = jax.lax.dot(xcat, G1[...], preferred_element_type=f32) + bz1[...]
    t = jnp.maximum(Z[:, 0:64], 0.0).astype(bf)
    P = jax.lax.dot(t, G2[...], preferred_element_type=f32)
    ea2_r[...] = (P[:, 0:64] + be2[...]).astype(bf)
    t2 = jnp.maximum(P[:, 64:128] + Z[:, 64:128] + bh[...], 0.0).astype(bf)
    h_r[...] = jax.lax.dot(t2, W2n[...], preferred_element_type=f32) + b2n[...]


def _tc_edge(xr, xc, ea, G1, bz1, G2, be2, bh, W2n, b2n, dx):
    E, Dg = xr.shape
    De = ea.shape[1]
    G = E // _BE
    grid_spec = pl.GridSpec(
        grid=(G,),
        in_specs=[
            pl.BlockSpec((_BE, Dg), lambda i: (i, 0)),
            pl.BlockSpec((_BE, Dg), lambda i: (i, 0)),
            pl.BlockSpec((_BE, De), lambda i: (i, 0)),
            _full(G1.shape), _full(bz1.shape), _full(G2.shape),
            _full(be2.shape), _full(bh.shape), _full(W2n.shape),
            _full(b2n.shape),
        ],
        out_specs=[
            pl.BlockSpec((_BE, 64), lambda i: (i, 0)),
            pl.BlockSpec((_BE, 64), lambda i: (i, 0)),
        ],
    )
    return pl.pallas_call(
        functools.partial(_edge_body, dx),
        grid_spec=grid_spec,
        out_shape=[
            jax.ShapeDtypeStruct((E, 64), jnp.bfloat16),
            jax.ShapeDtypeStruct((E, 64), jnp.float32),
        ],
        compiler_params=pltpu.CompilerParams(
            dimension_semantics=("arbitrary",)
        ),
    )(xr, xc, ea, G1, bz1, G2, be2, bh, W2n, b2n)


def _edge_final_body(xr_r, xc_r, ea_r, W1, b1, W2, b2, out_r):
    bf = jnp.bfloat16
    xcat = jnp.concatenate(
        [xr_r[...][:, 0:64], xc_r[...][:, 0:64], ea_r[...].astype(bf)],
        axis=1,
    )
    t = jax.lax.dot(xcat, W1[...], preferred_element_type=jnp.float32) + b1[...]
    t = jnp.maximum(t, 0.0).astype(bf)
    z = jax.lax.dot(t, W2[...], preferred_element_type=jnp.float32) + b2[...]
    out_r[...] = jax.nn.sigmoid(z)


def _tc_edge_final(xr, xc, ea, W1, b1, W2, b2):
    E, Dg = xr.shape
    De = ea.shape[1]
    G = E // _BE
    grid_spec = pl.GridSpec(
        grid=(G,),
        in_specs=[
            pl.BlockSpec((_BE, Dg), lambda i: (i, 0)),
            pl.BlockSpec((_BE, Dg), lambda i: (i, 0)),
            pl.BlockSpec((_BE, De), lambda i: (i, 0)),
            _full(W1.shape), _full(b1.shape),
            _full(W2.shape), _full(b2.shape),
        ],
        out_specs=[pl.BlockSpec((_BE, 1), lambda i: (i, 0))],
    )
    return pl.pallas_call(
        _edge_final_body,
        grid_spec=grid_spec,
        out_shape=[jax.ShapeDtypeStruct((E, 1), jnp.float32)],
        compiler_params=pltpu.CompilerParams(
            dimension_semantics=("arbitrary",)
        ),
    )(xr, xc, ea, W1, b1, W2, b2)[0]


def _node_body(x_r, sp_r, cp_r, V1, b1, W2, b2, out_r, outg_r):
    bf = jnp.bfloat16
    f32 = jnp.float32
    s = sp_r[0] + sp_r[1]
    c = cp_r[0, :, 0:1] + cp_r[1, :, 0:1]
    agg = s / jnp.maximum(c, 1.0)
    xcat = jnp.concatenate([x_r[...].astype(bf), agg.astype(bf)], axis=1)
    t = jax.lax.dot(xcat, V1[...], preferred_element_type=f32) + b1[...]
    t = jnp.maximum(t, 0.0).astype(bf)
    o = jax.lax.dot(t, W2[...], preferred_element_type=f32) + b2[...]
    out_r[...] = o
    outg_r[...] = jnp.concatenate(
        [o, jnp.zeros((o.shape[0], 64), f32)], axis=1
    )


def _tc_node(x, spart, cpart, V1, b1, W2, b2):
    N, Dx = x.shape
    G = N // _BN
    grid_spec = pl.GridSpec(
        grid=(G,),
        in_specs=[
            pl.BlockSpec((_BN, Dx), lambda i: (i, 0)),
            pl.BlockSpec((NC, _BN, 64), lambda i: (0, i, 0)),
            pl.BlockSpec((NC, _BN, 16), lambda i: (0, i, 0)),
            _full(V1.shape), _full(b1.shape),
            _full(W2.shape), _full(b2.shape),
        ],
        out_specs=[
            pl.BlockSpec((_BN, 64), lambda i: (i, 0)),
            pl.BlockSpec((_BN, 128), lambda i: (i, 0)),
        ],
    )
    return pl.pallas_call(
        _node_body,
        grid_spec=grid_spec,
        out_shape=[
            jax.ShapeDtypeStruct((N, 64), jnp.float32),
            jax.ShapeDtypeStruct((N, 128), jnp.float32),
        ],
        compiler_params=pltpu.CompilerParams(
            dimension_semantics=("arbitrary",)
        ),
    )(x, spart, cpart, V1, b1, W2, b2)


def _row(b):
    return b.reshape(1, -1)


def _pad_rows(W, rows_from, n_rows):
    sub = W[rows_from[0]:rows_from[1]]
    return jnp.pad(sub, ((0, n_rows - sub.shape[0]), (0, 0)))


def _prep_edge(pe, pn1, dx_raw, dx):
    bf = jnp.bfloat16
    W1e, b1e, W2e, b2e = pe
    W1n, b1n, W2n, b2n = pn1
    de_raw = W1e.shape[0] - 2 * dx_raw
    if dx_raw == dx:
        A, B = W1e[0:dx], W1e[dx:2 * dx]
    else:
        A = _pad_rows(W1e, (0, dx_raw), dx)
        B = _pad_rows(W1e, (dx_raw, 2 * dx_raw), dx)
    C = W1e[2 * dx_raw:]
    D1 = W1n[0:dx_raw]
    if dx_raw != dx:
        D1 = _pad_rows(W1n, (0, dx_raw), dx)
    Ew = W1n[dx_raw:dx_raw + 64]
    G1 = jnp.concatenate(
        [jnp.concatenate([A, B, C], axis=0),
         jnp.concatenate(
             [D1, jnp.zeros((dx + de_raw, 64), jnp.float32)], axis=0)],
        axis=1).astype(bf)
    bz1 = jnp.concatenate([b1e, jnp.zeros((64,), jnp.float32)]).reshape(1, 128)
    W2eEw = W2e @ Ew
    G2 = jnp.concatenate([W2e, W2eEw], axis=1).astype(bf)
    bh = _row(b2e @ Ew + b1n)
    return G1, bz1, G2, _row(b2e), bh, W2n.astype(bf), _row(b2n)


def _prep_node(pn2, dx_raw, dx):
    bf = jnp.bfloat16
    V1n, c1n, V2n, c2n = pn2
    Wx = V1n[0:dx_raw]
    if dx_raw != dx:
        Wx = _pad_rows(V1n, (0, dx_raw), dx)
    Wa = V1n[dx_raw:dx_raw + 64]
    V1 = jnp.concatenate([Wx, Wa], axis=0).astype(bf)
    return V1, _row(c1n), V2n.astype(bf), _row(c2n)


def kernel(x, edge_index, edge_attr, params):
    N, _ = x.shape
    E = edge_attr.shape[0]
    npad = _idx_pad_rows(E) - E // CH
    row = jnp.pad(edge_index[0].reshape(E // CH, CH), ((0, npad), (0, 0)))
    col = jnp.pad(edge_index[1].reshape(E // CH, CH), ((0, npad), (0, 0)))


    bf = jnp.bfloat16
    x16 = jnp.pad(x, ((0, 0), (0, 14)))
    xg = jnp.pad(x, ((0, 0), (0, 126)))
    zra = (N // NS // 8) * 8
    ztail = zra + N - zra * NS
    zeros64 = jnp.zeros((ztail, 64), jnp.float32)
    zeros16 = jnp.zeros((ztail, 16), jnp.float32)
    ones16 = jnp.ones((E, 16), jnp.float32)

    cpart = _scatter_add(E, 16, N)(ones16, col, zeros16)

    ew = _prep_edge(params['c1_e'], params['c1_n1'], 2, 16)
    xr, xc = _gather2(N, 128, E)(xg, row, col)
    ea, h = _tc_edge(xr, xc, edge_attr, *ew, dx=16)
    spart = _scatter_add(E, 64, N)(h, col, zeros64)
    nw = _prep_node(params['c1_n2'], 2, 16)
    xl, xlg = _tc_node(x16, spart, cpart, *nw)

    for name in ('c2', 'c3', 'c4'):
        ew = _prep_edge(params[name + '_e'], params[name + '_n1'], 64, 64)
        xr, xc = _gather2(N, 128, E)(xlg, row, col)
        ea, h = _tc_edge(xr, xc, ea, *ew, dx=64)
        spart = _scatter_add(E, 64, N)(h, col, zeros64)
        nw = _prep_node(params[name + '_n2'], 64, 64)
        xl, xlg = _tc_node(xl, spart, cpart, *nw)

    W1f, b1f, W2f, b2f = params['c5_e']
    xr, xc = _gather2(N, 128, E)(xlg, row, col)
    return _tc_edge_final(xr, xc, ea, W1f.astype(bf), _row(b1f),
                          W2f.astype(bf), _row(b2f).reshape(1, 1))


# speedup vs baseline: 1.8637x; 1.0003x over previous
import functools

import jax
import jax.numpy as jnp
from jax import lax
from jax.experimental import pallas as pl
from jax.experimental.pallas import tpu as pltpu
from jax.experimental.pallas import tpu_sc as plsc

NC, NS, L = 2, 16, 16
NW = NC * NS
CH = 128


GRP = 3


def _tile_rows(E):
    nchk = E // CH
    base = nchk // NW
    extra = nchk - base * NW
    return nchk, base, extra


def _idx_pad_rows(E):
    nchk, base, extra = _tile_rows(E)
    maxrow0 = base * (NW - 1) + extra
    return (maxrow0 // 8) * 8 + (base + (1 if extra else 0)) + 8


def _row_start(wid, base, extra):
    return jnp.where(
        wid < extra, (base + 1) * wid, extra * (base + 1) + base * (wid - extra)
    )


@functools.lru_cache(maxsize=None)
def _gather2(N, D, E, dtype=jnp.float32):
    nchk, base, extra = _tile_rows(E)
    maxrows = base + (1 if extra else 0)
    nfullg = base // GRP
    rem = base - nfullg * GRP
    pre = maxrows + 8
    nchkp = _idx_pad_rows(E)
    mesh = plsc.VectorSubcoreMesh(core_axis_name="c", subcore_axis_name="s")

    @functools.partial(
        pl.kernel,
        mesh=mesh,
        out_type=(
            jax.ShapeDtypeStruct((E, D), dtype),
            jax.ShapeDtypeStruct((E, D), dtype),
        ),
        scratch_types=[
            pltpu.VMEM((pre, CH), jnp.int32),
            pltpu.VMEM((GRP * CH, D), dtype),
            pltpu.VMEM((GRP * CH, D), dtype),
            pltpu.SemaphoreType.DMA,
            pltpu.SemaphoreType.DMA,
        ],
    )
    def k(table_hbm, row_hbm, col_hbm, xr_hbm, xc_hbm, idx_v, bufa, bufb, gsem, osem):
        wid = lax.axis_index("s") * NC + lax.axis_index("c")
        row0 = _row_start(wid, base, extra)
        row0a = pl.multiple_of((row0 // 8) * 8, 8)
        delta = row0 - row0a
        has_extra = wid < extra
        bufs = (bufa, bufb)

        def do(idx2d_hbm, out_hbm):
            pltpu.sync_copy(idx2d_hbm.at[pl.ds(row0a, pre)], idx_v)

            groups = []
            for g in range(nfullg):
                groups.append((g * GRP, GRP, False))
            if rem or extra:
                groups.append((nfullg * GRP, rem, bool(extra)))

            out_desc = [None, None]
            for gi, (r0, nr, cond) in enumerate(groups):
                buf = bufs[gi % 2]
                if out_desc[gi % 2] is not None and nr:
                    out_desc[gi % 2].wait()
                descs = []
                for j in range(nr):
                    descs.append(
                        pltpu.async_copy(
                            table_hbm.at[idx_v.at[delta + (r0 + j)]],
                            buf.at[pl.ds(j * CH, CH)],
                            gsem,
                        )
                    )
                if cond:
                    @pl.when(has_extra)
                    def _(r0=r0, nr=nr, buf=buf):
                        pltpu.async_copy(
                            table_hbm.at[idx_v.at[delta + (r0 + nr)]],
                            buf.at[pl.ds(nr * CH, CH)],
                            gsem,
                        ).wait()
                for d in descs:
                    d.wait()
                off = (row0 + r0) * CH
                if nr:
                    out_desc[gi % 2] = pltpu.async_copy(
                        buf.at[pl.ds(0, nr * CH)],
                        out_hbm.at[pl.ds(off, nr * CH)],
                        osem,
                    )
                if cond:
                    @pl.when(has_extra)
                    def _(r0=r0, nr=nr, buf=buf):
                        pltpu.async_copy(
                            buf.at[pl.ds(nr * CH, CH)],
                            out_hbm.at[pl.ds((row0 + r0 + nr) * CH, CH)],
                            osem,
                        ).wait()
            for d in out_desc:
                if d is not None:
                    d.wait()

        do(row_hbm, xr_hbm)
        do(col_hbm, xc_hbm)

    return k


@functools.lru_cache(maxsize=None)
def _scatter_add(E, D, N):
    nchk, base, extra = _tile_rows(E)
    maxrows = base + (1 if extra else 0)
    nfullg = base // GRP
    rem = base - nfullg * GRP
    pre = maxrows + 8
    ZR = (N // NS // 8) * 8
    ZLAST = N - ZR * (NS - 1)
    mesh = plsc.VectorSubcoreMesh(core_axis_name="c", subcore_axis_name="s")

    @functools.partial(
        pl.kernel,
        mesh=mesh,
        out_type=jax.ShapeDtypeStruct((NC, N, D), jnp.float32),
        scratch_types=[
            pltpu.VMEM((pre, CH), jnp.int32),
            pltpu.VMEM((GRP * CH, D), jnp.float32),
            pltpu.VMEM((GRP * CH, D), jnp.float32),
            pltpu.VMEM_SHARED((N, D), jnp.float32),
            pltpu.SemaphoreType.DMA,
        ],
        compiler_params=pltpu.CompilerParams(use_tc_tiling_on_sc=False),
    )
    def k(vals_hbm, idx_hbm, zeros_hbm, out_hbm, idx_v, bufa, bufb, acc, lsem):
        cid = lax.axis_index("c")
        sid = lax.axis_index("s")
        wid = sid * NC + cid
        row0 = _row_start(wid, base, extra)
        row0a = pl.multiple_of((row0 // 8) * 8, 8)
        delta = row0 - row0a
        has_extra = wid < extra
        bufs = (bufa, bufb)

        z0 = pl.multiple_of(sid * ZR, 8)
        zn = jnp.where(sid == NS - 1, ZLAST, ZR)

        @pl.when(sid < NS - 1)
        def _():
            pltpu.sync_copy(
                zeros_hbm.at[pl.ds(0, ZR)], acc.at[pl.ds(z0, ZR)]
            )

        @pl.when(sid == NS - 1)
        def _():
            pltpu.sync_copy(zeros_hbm, acc.at[pl.ds(z0, ZLAST)])

        pltpu.sync_copy(idx_hbm.at[pl.ds(row0a, pre)], idx_v)
        plsc.subcore_barrier()

        groups = []
        for g in range(nfullg):
            groups.append((g * GRP, GRP, False))
        if rem or extra:
            groups.append((nfullg * GRP, rem, bool(extra)))

        def load(r0, nr, cond, buf):
            dx = []
            if nr:
                dx.append(pltpu.async_copy(
                    vals_hbm.at[pl.ds((row0 + r0) * CH, nr * CH)],
                    buf.at[pl.ds(0, nr * CH)],
                    lsem,
                ))
            if cond:
                @pl.when(has_extra)
                def _():
                    pltpu.async_copy(
                        vals_hbm.at[pl.ds((row0 + r0 + nr) * CH, CH)],
                        buf.at[pl.ds(nr * CH, CH)],
                        lsem,
                    ).wait()
            return dx

        descs = {}
        descs[0] = load(*groups[0], bufs[0])
        if len(groups) > 1:
            descs[1] = load(*groups[1], bufs[1])
        for gi, (r0, nr, cond) in enumerate(groups):
            buf = bufs[gi % 2]
            for d in descs.pop(gi):
                d.wait()
            for j in range(nr):
                pltpu.sync_copy(
                    buf.at[pl.ds(j * CH, CH)],
                    acc.at[idx_v.at[delta + (r0 + j)]],
                    add=True,
                )
            if cond:
                @pl.when(has_extra)
                def _(r0=r0, nr=nr, buf=buf):
                    pltpu.sync_copy(
                        buf.at[pl.ds(nr * CH, CH)],
                        acc.at[idx_v.at[delta + (r0 + nr)]],
                        add=True,
                    )
            if gi + 2 < len(groups):
                descs[gi + 2] = load(*groups[gi + 2], buf)

        plsc.subcore_barrier()

        @pl.when(sid < NS - 1)
        def _():
            pltpu.sync_copy(
                acc.at[pl.ds(z0, ZR)], out_hbm.at[cid, pl.ds(z0, ZR)]
            )

        @pl.when(sid == NS - 1)
        def _():
            pltpu.sync_copy(
                acc.at[pl.ds(z0, ZLAST)], out_hbm.at[cid, pl.ds(z0, ZLAST)]
            )

    return k


@functools.lru_cache(maxsize=None)
def _count_scatter(E, N):
    nchk, base, extra = _tile_rows(E)
    maxrows = base + (1 if extra else 0)
    pre = maxrows + 8
    ZR = (N // NS // 8) * 8
    ZLAST = N - ZR * (NS - 1)
    mesh = plsc.VectorSubcoreMesh(core_axis_name="c", subcore_axis_name="s")

    @functools.partial(
        pl.kernel,
        mesh=mesh,
        out_type=jax.ShapeDtypeStruct((NC, N, 16), jnp.float32),
        scratch_types=[
            pltpu.VMEM((pre, CH), jnp.int32),
            pltpu.VMEM((CH, 16), jnp.float32),
            pltpu.VMEM_SHARED((N, 16), jnp.float32),
        ],
        compiler_params=pltpu.CompilerParams(use_tc_tiling_on_sc=False),
    )
    def k(idx_hbm, zeros_hbm, out_hbm, idx_v, ones_v, acc):
        cid = lax.axis_index("c")
        sid = lax.axis_index("s")
        wid = sid * NC + cid
        row0 = _row_start(wid, base, extra)
        row0a = pl.multiple_of((row0 // 8) * 8, 8)
        delta = row0 - row0a
        has_extra = wid < extra

        z0 = pl.multiple_of(sid * ZR, 8)

        @pl.when(sid < NS - 1)
        def _():
            pltpu.sync_copy(zeros_hbm.at[pl.ds(0, ZR)], acc.at[pl.ds(z0, ZR)])

        @pl.when(sid == NS - 1)
        def _():
            pltpu.sync_copy(zeros_hbm, acc.at[pl.ds(z0, ZLAST)])

        pltpu.sync_copy(idx_hbm.at[pl.ds(row0a, pre)], idx_v)

        @pl.loop(0, CH)
        def _(i):
            ones_v[i, pl.ds(0, 16)] = jnp.ones((16,), jnp.float32)

        plsc.subcore_barrier()

        @pl.loop(0, base)
        def _(r):
            pltpu.sync_copy(ones_v, acc.at[idx_v.at[delta + r]], add=True)

        if extra:
            @pl.when(has_extra)
            def _():
                pltpu.sync_copy(
                    ones_v, acc.at[idx_v.at[delta + base]], add=True
                )

        plsc.subcore_barrier()

        @pl.when(sid < NS - 1)
        def _():
            pltpu.sync_copy(
                acc.at[pl.ds(z0, ZR)], out_hbm.at[cid, pl.ds(z0, ZR)]
            )

        @pl.when(sid == NS - 1)
        def _():
            pltpu.sync_copy(
                acc.at[pl.ds(z0, ZLAST)], out_hbm.at[cid, pl.ds(z0, ZLAST)]
            )

    return k


_BE = 4000
_BN = 2000


def _full(shape):
    return pl.BlockSpec(shape, lambda i: (0,) * len(shape))


def _edge_body(dx, xr_r, xc_r, ea_r, G1, bz1, G2, be2, bh, W2n, b2n,
               ea2_r, h_r):
    bf = jnp.bfloat16
    f32 = jnp.float32
    xcat = jnp.concatenate(
        [xr_r[...][:, 0:dx].astype(bf), xc_r[...][:, 0:dx].astype(bf),
         ea_r[...].astype(bf)],
        axis=1,
    )
    Z = jax.lax.dot(xcat, G1[...], preferred_element_type=f32) + bz1[...]
    t = jnp.maximum(Z[:, 0:64], 0.0).astype(bf)
    P = jax.lax.dot(t, G2[...], preferred_element_type=f32)
    ea2_r[...] = (P[:, 0:64] + be2[...]).astype(bf)
    t2 = jnp.maximum(P[:, 64:128] + Z[:, 64:128] + bh[...], 0.0).astype(bf)
    h_r[...] = jax.lax.dot(t2, W2n[...], preferred_element_type=f32) + b2n[...]


def _tc_edge(xr, xc, ea, G1, bz1, G2, be2, bh, W2n, b2n, dx):
    E, Dg = xr.shape
    De = ea.shape[1]
    G = E // _BE
    grid_spec = pl.GridSpec(
        grid=(G,),
        in_specs=[
            pl.BlockSpec((_BE, Dg), lambda i: (i, 0)),
            pl.BlockSpec((_BE, Dg), lambda i: (i, 0)),
            pl.BlockSpec((_BE, De), lambda i: (i, 0)),
            _full(G1.shape), _full(bz1.shape), _full(G2.shape),
            _full(be2.shape), _full(bh.shape), _full(W2n.shape),
            _full(b2n.shape),
        ],
        out_specs=[
            pl.BlockSpec((_BE, 64), lambda i: (i, 0)),
            pl.BlockSpec((_BE, 64), lambda i: (i, 0)),
        ],
    )
    return pl.pallas_call(
        functools.partial(_edge_body, dx),
        grid_spec=grid_spec,
        out_shape=[
            jax.ShapeDtypeStruct((E, 64), jnp.bfloat16),
            jax.ShapeDtypeStruct((E, 64), jnp.float32),
        ],
        compiler_params=pltpu.CompilerParams(
            dimension_semantics=("arbitrary",)
        ),
    )(xr, xc, ea, G1, bz1, G2, be2, bh, W2n, b2n)


def _edge_final_body(xr_r, xc_r, ea_r, W1, b1, W2, b2, out_r):
    bf = jnp.bfloat16
    xcat = jnp.concatenate(
        [xr_r[...][:, 0:64], xc_r[...][:, 0:64], ea_r[...].astype(bf)],
        axis=1,
    )
    t = jax.lax.dot(xcat, W1[...], preferred_element_type=jnp.float32) + b1[...]
    t = jnp.maximum(t, 0.0).astype(bf)
    z = jax.lax.dot(t, W2[...], preferred_element_type=jnp.float32) + b2[...]
    out_r[...] = jax.nn.sigmoid(z)


def _tc_edge_final(xr, xc, ea, W1, b1, W2, b2):
    E, Dg = xr.shape
    De = ea.shape[1]
    G = E // _BE
    grid_spec = pl.GridSpec(
        grid=(G,),
        in_specs=[
            pl.BlockSpec((_BE, Dg), lambda i: (i, 0)),
            pl.BlockSpec((_BE, Dg), lambda i: (i, 0)),
            pl.BlockSpec((_BE, De), lambda i: (i, 0)),
            _full(W1.shape), _full(b1.shape),
            _full(W2.shape), _full(b2.shape),
        ],
        out_specs=[pl.BlockSpec((_BE, 1), lambda i: (i, 0))],
    )
    return pl.pallas_call(
        _edge_final_body,
        grid_spec=grid_spec,
        out_shape=[jax.ShapeDtypeStruct((E, 1), jnp.float32)],
        compiler_params=pltpu.CompilerParams(
            dimension_semantics=("arbitrary",)
        ),
    )(xr, xc, ea, W1, b1, W2, b2)[0]


def _node_body(x_r, sp_r, cp_r, V1, b1, W2, b2, out_r, outg_r):
    bf = jnp.bfloat16
    f32 = jnp.float32
    s = sp_r[0] + sp_r[1]
    c = cp_r[0, :, 0:1] + cp_r[1, :, 0:1]
    agg = s / jnp.maximum(c, 1.0)
    xcat = jnp.concatenate([x_r[...].astype(bf), agg.astype(bf)], axis=1)
    t = jax.lax.dot(xcat, V1[...], preferred_element_type=f32) + b1[...]
    t = jnp.maximum(t, 0.0).astype(bf)
    o = jax.lax.dot(t, W2[...], preferred_element_type=f32) + b2[...]
    out_r[...] = o
    outg_r[...] = jnp.concatenate(
        [o, jnp.zeros((o.shape[0], 64), f32)], axis=1
    )


def _tc_node(x, spart, cpart, V1, b1, W2, b2):
    N, Dx = x.shape
    G = N // _BN
    grid_spec = pl.GridSpec(
        grid=(G,),
        in_specs=[
            pl.BlockSpec((_BN, Dx), lambda i: (i, 0)),
            pl.BlockSpec((NC, _BN, 64), lambda i: (0, i, 0)),
            pl.BlockSpec((NC, _BN, 16), lambda i: (0, i, 0)),
            _full(V1.shape), _full(b1.shape),
            _full(W2.shape), _full(b2.shape),
        ],
        out_specs=[
            pl.BlockSpec((_BN, 64), lambda i: (i, 0)),
            pl.BlockSpec((_BN, 128), lambda i: (i, 0)),
        ],
    )
    return pl.pallas_call(
        _node_body,
        grid_spec=grid_spec,
        out_shape=[
            jax.ShapeDtypeStruct((N, 64), jnp.float32),
            jax.ShapeDtypeStruct((N, 128), jnp.float32),
        ],
        compiler_params=pltpu.CompilerParams(
            dimension_semantics=("arbitrary",)
        ),
    )(x, spart, cpart, V1, b1, W2, b2)


def _row(b):
    return b.reshape(1, -1)


def _pad_rows(W, rows_from, n_rows):
    sub = W[rows_from[0]:rows_from[1]]
    return jnp.pad(sub, ((0, n_rows - sub.shape[0]), (0, 0)))


def _prep_edge(pe, pn1, dx_raw, dx):
    bf = jnp.bfloat16
    W1e, b1e, W2e, b2e = pe
    W1n, b1n, W2n, b2n = pn1
    de_raw = W1e.shape[0] - 2 * dx_raw
    if dx_raw == dx:
        A, B = W1e[0:dx], W1e[dx:2 * dx]
    else:
        A = _pad_rows(W1e, (0, dx_raw), dx)
        B = _pad_rows(W1e, (dx_raw, 2 * dx_raw), dx)
    C = W1e[2 * dx_raw:]
    D1 = W1n[0:dx_raw]
    if dx_raw != dx:
        D1 = _pad_rows(W1n, (0, dx_raw), dx)
    Ew = W1n[dx_raw:dx_raw + 64]
    G1 = jnp.concatenate(
        [jnp.concatenate([A, B, C], axis=0),
         jnp.concatenate(
             [D1, jnp.zeros((dx + de_raw, 64), jnp.float32)], axis=0)],
        axis=1).astype(bf)
    bz1 = jnp.concatenate([b1e, jnp.zeros((64,), jnp.float32)]).reshape(1, 128)
    W2eEw = W2e @ Ew
    G2 = jnp.concatenate([W2e, W2eEw], axis=1).astype(bf)
    bh = _row(b2e @ Ew + b1n)
    return G1, bz1, G2, _row(b2e), bh, W2n.astype(bf), _row(b2n)


def _prep_node(pn2, dx_raw, dx):
    bf = jnp.bfloat16
    V1n, c1n, V2n, c2n = pn2
    Wx = V1n[0:dx_raw]
    if dx_raw != dx:
        Wx = _pad_rows(V1n, (0, dx_raw), dx)
    Wa = V1n[dx_raw:dx_raw + 64]
    V1 = jnp.concatenate([Wx, Wa], axis=0).astype(bf)
    return V1, _row(c1n), V2n.astype(bf), _row(c2n)


def kernel(x, edge_index, edge_attr, params):
    N, _ = x.shape
    E = edge_attr.shape[0]
    npad = _idx_pad_rows(E) - E // CH
    row = jnp.pad(edge_index[0].reshape(E // CH, CH), ((0, npad), (0, 0)))
    col = jnp.pad(edge_index[1].reshape(E // CH, CH), ((0, npad), (0, 0)))


    bf = jnp.bfloat16
    x16 = jnp.pad(x, ((0, 0), (0, 14)))
    xg = jnp.pad(x, ((0, 0), (0, 126)))
    zra = (N // NS // 8) * 8
    ztail = zra + N - zra * NS
    zeros64 = jnp.zeros((ztail, 64), jnp.float32)
    zeros16 = jnp.zeros((ztail, 16), jnp.float32)

    cpart = _count_scatter(E, N)(col, zeros16)

    ew = _prep_edge(params['c1_e'], params['c1_n1'], 2, 16)
    xr, xc = _gather2(N, 128, E)(xg, row, col)
    ea, h = _tc_edge(xr, xc, edge_attr, *ew, dx=16)
    spart = _scatter_add(E, 64, N)(h, col, zeros64)
    nw = _prep_node(params['c1_n2'], 2, 16)
    xl, xlg = _tc_node(x16, spart, cpart, *nw)

    for name in ('c2', 'c3', 'c4'):
        ew = _prep_edge(params[name + '_e'], params[name + '_n1'], 64, 64)
        xr, xc = _gather2(N, 128, E)(xlg, row, col)
        ea, h = _tc_edge(xr, xc, ea, *ew, dx=64)
        spart = _scatter_add(E, 64, N)(h, col, zeros64)
        nw = _prep_node(params[name + '_n2'], 64, 64)
        xl, xlg = _tc_node(xl, spart, cpart, *nw)

    W1f, b1f, W2f, b2f = params['c5_e']
    xr, xc = _gather2(N, 128, E)(xlg, row, col)
    return _tc_edge_final(xr, xc, ea, W1f.astype(bf), _row(b1f),
                          W2f.astype(bf), _row(b2f).reshape(1, 1))
